# baseline (speedup 1.0000x reference)
import functools
import numpy as np
import jax
import jax.numpy as jnp
from jax import lax
from jax.experimental import pallas as pl
from jax.experimental.pallas import tpu as pltpu
from jax.experimental.pallas import tpu_sc as plsc

_SCALES = (0.25, 0.5, 1.0, 2.0)
_ASPECT_RATIOS = (0.5, 1.0, 2.0)
_ANCHOR_STRIDE = (16, 16)
_MAX_PROPOSALS = 100
_NMS_IOU_THRESHOLD = 0.699999988079
_BASE_ANCHOR_SIZE = 256.0

_N = 12288
_NSUB = 16
_PER = _N // _NSUB
_NJ = _PER // 16


def _anchor_vecs(Hf, Wf):
    ys = (np.arange(Hf, dtype=np.float32) + 0.5) * _ANCHOR_STRIDE[0]
    xs = (np.arange(Wf, dtype=np.float32) + 0.5) * _ANCHOR_STRIDE[1]
    sc, ar = np.meshgrid(np.array(_SCALES, np.float32),
                         np.array(_ASPECT_RATIOS, np.float32), indexing='ij')
    sc = sc.reshape(-1)
    ar = ar.reshape(-1)
    ha = sc * _BASE_ANCHOR_SIZE / np.sqrt(ar)
    wa = sc * _BASE_ANCHOR_SIZE * np.sqrt(ar)
    A = ha.shape[0]
    yy, xx = np.meshgrid(ys, xs, indexing='ij')
    ycent = np.repeat(yy.reshape(-1), A)
    xcent = np.repeat(xx.reshape(-1), A)
    hh = np.tile(ha, Hf * Wf)
    ww = np.tile(wa, Hf * Wf)
    return (jnp.asarray(ycent), jnp.asarray(xcent),
            jnp.asarray(hh), jnp.asarray(ww))


def _sc_nms_body(ench, clsh, yah, xah, hah, wah, outh,
                 enc_v, cls_v, ya_v, xa_v, ha_v, wa_v,
                 by_ref, bx_ref, ey_ref, ex_ref, ar_ref, sc_ref,
                 pub_ref, allc_ref, outv_ref, shared_ref):
    cid = lax.axis_index("c")
    sid = lax.axis_index("s")

    @pl.when(cid == 0)
    def _core0():
        base = sid * _PER
        lane = lax.iota(jnp.int32, 16)
        thr = jnp.float32(_NMS_IOU_THRESHOLD)
        neg = jnp.float32(-1e9)
        inv = jnp.float32(1.0 / 512.0)
        big = jnp.int32(2 ** 30)
        sidv = jnp.broadcast_to(sid, (16,))
        negv = jnp.broadcast_to(neg, (16,))

        pltpu.sync_copy(ench.at[pl.ds(base * 4, _PER * 4)], enc_v)
        pltpu.sync_copy(clsh.at[pl.ds(base * 2, _PER * 2)], cls_v)
        pltpu.sync_copy(yah.at[pl.ds(base, _PER)], ya_v)
        pltpu.sync_copy(xah.at[pl.ds(base, _PER)], xa_v)
        pltpu.sync_copy(hah.at[pl.ds(base, _PER)], ha_v)
        pltpu.sync_copy(wah.at[pl.ds(base, _PER)], wa_v)

        lane4 = lane * 4
        lane2 = lane * 2

        bv = jnp.full((16,), -jnp.inf, jnp.float32)
        bj = jnp.zeros((16,), jnp.int32)
        for j in range(_NJ):
            sl = pl.ds(j * 16, 16)
            ty = plsc.load_gather(enc_v, [lane4 + j * 64]) / 10.0
            tx = plsc.load_gather(enc_v, [lane4 + (j * 64 + 1)]) / 10.0
            th = plsc.load_gather(enc_v, [lane4 + (j * 64 + 2)]) / 5.0
            tw = plsc.load_gather(enc_v, [lane4 + (j * 64 + 3)]) / 5.0
            ya = ya_v[sl]
            xa = xa_v[sl]
            ha = ha_v[sl]
            wa = wa_v[sl]
            yc = ty * ha + ya
            xc = tx * wa + xa
            h = jnp.exp(th) * ha
            w = jnp.exp(tw) * wa
            ymin = jnp.clip(yc - h / 2.0, 0.0, 512.0)
            xmin = jnp.clip(xc - w / 2.0, 0.0, 512.0)
            ymax = jnp.clip(yc + h / 2.0, 0.0, 512.0)
            xmax = jnp.clip(xc + w / 2.0, 0.0, 512.0)
            cb = plsc.load_gather(cls_v, [lane2 + j * 32])
            cf = plsc.load_gather(cls_v, [lane2 + (j * 32 + 1)])
            mx = jnp.maximum(cb, cf)
            eb = jnp.exp(cb - mx)
            ef = jnp.exp(cf - mx)
            s = ef / (eb + ef)
            by_ref[sl] = ymin
            bx_ref[sl] = xmin
            ey_ref[sl] = ymax
            ex_ref[sl] = xmax
            ar_ref[sl] = (jnp.maximum(ymax - ymin, 0.0)
                          * jnp.maximum(xmax - xmin, 0.0))
            sc_ref[sl] = s
            better = s > bv
            bv = jnp.where(better, s, bv)
            bj = jnp.where(better, jnp.broadcast_to(j, (16,)), bj)

        def step(t, carry):
            bv, bj = carry
            bi = lane + (base + bj * 16)
            m_loc = jnp.max(bv)
            gl = jnp.min(jnp.where(bv == m_loc, bi, big))
            idxv = jnp.broadcast_to(gl - base, (16,))
            y0 = plsc.load_gather(by_ref, [idxv])
            x0 = plsc.load_gather(bx_ref, [idxv])
            y1 = plsc.load_gather(ey_ref, [idxv])
            x1 = plsc.load_gather(ex_ref, [idxv])
            pub = (jnp.where(lane == 0, m_loc, 0.0)
                   + jnp.where(lane == 1, y0, 0.0)
                   + jnp.where(lane == 2, x0, 0.0)
                   + jnp.where(lane == 3, y1, 0.0)
                   + jnp.where(lane == 4, x1, 0.0)
                   + jnp.where(lane == 5, gl.astype(jnp.float32), 0.0))
            pub_ref[...] = pub
            slot = lax.rem(t, 2)
            pltpu.sync_copy(pub_ref,
                            shared_ref.at[pl.ds(slot * 256 + sid * 16, 16)])
            plsc.subcore_barrier()
            pltpu.sync_copy(shared_ref.at[pl.ds(slot * 256, 256)], allc_ref)

            s16 = plsc.load_gather(allc_ref, [lane * 16])
            m = jnp.max(s16)
            r16 = jnp.broadcast_to(plsc.all_reduce_ffs(s16 == m), (16,))
            fi = r16 * 16
            by0 = plsc.load_gather(allc_ref, [fi + 1])
            by1 = plsc.load_gather(allc_ref, [fi + 2])
            by2 = plsc.load_gather(allc_ref, [fi + 3])
            by3 = plsc.load_gather(allc_ref, [fi + 4])
            g16 = plsc.load_gather(allc_ref, [fi + 5]).astype(jnp.int32)

            validv = jnp.broadcast_to(m, (16,)) > 0.0
            row = (jnp.where(lane == 0, by0, 0.0)
                   + jnp.where(lane == 1, by1, 0.0)
                   + jnp.where(lane == 2, by2, 0.0)
                   + jnp.where(lane == 3, by3, 0.0))
            row = jnp.where(validv, row, 0.0) * inv
            outv_ref[pl.ds(t * 16, 16)] = row

            killmask = (r16 == sidv) & (lane == 0)
            kidx = jnp.clip(g16 - jnp.broadcast_to(base, (16,)), 0, _PER - 1)
            plsc.store_scatter(sc_ref, [kidx], negv, mask=killmask)

            area_a = (jnp.maximum(by2 - by0, 0.0)
                      * jnp.maximum(by3 - by1, 0.0))
            nbv = jnp.full((16,), -jnp.inf, jnp.float32)
            nbj = jnp.zeros((16,), jnp.int32)
            for j in range(_NJ):
                sl = pl.ds(j * 16, 16)
                iy1 = jnp.maximum(by0, by_ref[sl])
                ix1 = jnp.maximum(by1, bx_ref[sl])
                iy2 = jnp.minimum(by2, ey_ref[sl])
                ix2 = jnp.minimum(by3, ex_ref[sl])
                inter = (jnp.maximum(iy2 - iy1, 0.0)
                         * jnp.maximum(ix2 - ix1, 0.0))
                union = area_a + ar_ref[sl] - inter
                iou = inter / jnp.maximum(union, 1e-8)
                ns = jnp.where(iou > thr, negv, sc_ref[sl])
                sc_ref[sl] = ns
                better = ns > nbv
                nbv = jnp.where(better, ns, nbv)
                nbj = jnp.where(better, jnp.broadcast_to(j, (16,)), nbj)
            return (nbv, nbj)

        lax.fori_loop(0, _MAX_PROPOSALS, step, (bv, bj))

        @pl.when(sid == 0)
        def _write_out():
            pltpu.sync_copy(outv_ref, outh)


def kernel(preprocessed_inputs, box_encodings, class_predictions_with_background,
           rpn_box_predictor_features, rpn_features_to_crop):
    del preprocessed_inputs, rpn_box_predictor_features, rpn_features_to_crop
    enc = box_encodings[0]
    cls = class_predictions_with_background[0]
    ya, xa, ha, wa = _anchor_vecs(32, 32)
    f32 = jnp.float32

    mesh = plsc.VectorSubcoreMesh(core_axis_name="c", subcore_axis_name="s")
    run = functools.partial(
        pl.kernel,
        mesh=mesh,
        compiler_params=pltpu.CompilerParams(needs_layout_passes=False),
        out_type=jax.ShapeDtypeStruct((_MAX_PROPOSALS * 16,), f32),
        scratch_types=(
            [pltpu.VMEM((_PER * 4,), f32), pltpu.VMEM((_PER * 2,), f32)]
            + [pltpu.VMEM((_PER,), f32) for _ in range(4)]
            + [pltpu.VMEM((_PER,), f32) for _ in range(6)]
            + [pltpu.VMEM((16,), f32),
               pltpu.VMEM((256,), f32),
               pltpu.VMEM((_MAX_PROPOSALS * 16,), f32),
               pltpu.VMEM_SHARED((512,), f32)]),
    )
    out = run(_sc_nms_body)(enc.reshape(_N * 4), cls.reshape(_N * 2),
                            ya, xa, ha, wa)
    return out.reshape(_MAX_PROPOSALS, 16)[:, :4][None]


# speedup vs baseline: 1.2054x; 1.2054x over previous
import functools
import numpy as np
import jax
import jax.numpy as jnp
from jax import lax
from jax.experimental import pallas as pl
from jax.experimental.pallas import tpu as pltpu
from jax.experimental.pallas import tpu_sc as plsc

_SCALES = (0.25, 0.5, 1.0, 2.0)
_ASPECT_RATIOS = (0.5, 1.0, 2.0)
_ANCHOR_STRIDE = (16, 16)
_MAX_PROPOSALS = 100
_NMS_IOU_THRESHOLD = 0.699999988079
_BASE_ANCHOR_SIZE = 256.0

_N = 12288
_ROWS, _COLS = 96, 128
_NSUB = 16
_PER = _N // _NSUB
_NJ = _PER // 16


def _anchor_planes(Hf, Wf):
    ys = (np.arange(Hf, dtype=np.float32) + 0.5) * _ANCHOR_STRIDE[0]
    xs = (np.arange(Wf, dtype=np.float32) + 0.5) * _ANCHOR_STRIDE[1]
    sc, ar = np.meshgrid(np.array(_SCALES, np.float32),
                         np.array(_ASPECT_RATIOS, np.float32), indexing='ij')
    sc = sc.reshape(-1)
    ar = ar.reshape(-1)
    ha = sc * _BASE_ANCHOR_SIZE / np.sqrt(ar)
    wa = sc * _BASE_ANCHOR_SIZE * np.sqrt(ar)
    A = ha.shape[0]
    yy, xx = np.meshgrid(ys, xs, indexing='ij')
    ycent = np.repeat(yy.reshape(-1), A)
    xcent = np.repeat(xx.reshape(-1), A)
    hh = np.tile(ha, Hf * Wf)
    ww = np.tile(wa, Hf * Wf)
    shape = (_ROWS, _COLS)
    return (jnp.asarray(ycent.reshape(shape)), jnp.asarray(xcent.reshape(shape)),
            jnp.asarray(hh.reshape(shape)), jnp.asarray(ww.reshape(shape)))


def _decode_body(tyr, txr, thr_, twr, cbr, cfr, yar, xar, har, war,
                 ymin_o, xmin_o, ymax_o, xmax_o, area_o, sc_o):
    ya = yar[:]
    xa = xar[:]
    ha = har[:]
    wa = war[:]
    ty = tyr[:] / 10.0
    tx = txr[:] / 10.0
    th = thr_[:] / 5.0
    tw = twr[:] / 5.0
    ycenter = ty * ha + ya
    xcenter = tx * wa + xa
    h = jnp.exp(th) * ha
    w = jnp.exp(tw) * wa
    ymin = jnp.clip(ycenter - h / 2.0, 0.0, 512.0)
    xmin = jnp.clip(xcenter - w / 2.0, 0.0, 512.0)
    ymax = jnp.clip(ycenter + h / 2.0, 0.0, 512.0)
    xmax = jnp.clip(xcenter + w / 2.0, 0.0, 512.0)
    cb = cbr[:]
    cf = cfr[:]
    mx = jnp.maximum(cb, cf)
    eb = jnp.exp(cb - mx)
    ef = jnp.exp(cf - mx)
    ymin_o[...] = ymin
    xmin_o[...] = xmin
    ymax_o[...] = ymax
    xmax_o[...] = xmax
    area_o[...] = (jnp.maximum(ymax - ymin, 0.0)
                   * jnp.maximum(xmax - xmin, 0.0))
    sc_o[...] = ef / (eb + ef)


def _sc_nms_body(tyh, txh, thh, twh, cbh, cfh, yah, xah, hah, wah, outh,
                 s0, s1, s2, s3, s4, s5, s6, s7, s8, s9,
                 by_ref, bx_ref, ey_ref, ex_ref, ar_ref, sc_ref,
                 pub_ref, allc_ref, outv_ref, shared_ref):
    cid = lax.axis_index("c")
    sid = lax.axis_index("s")

    @pl.when(cid == 0)
    def _core0():
        base = sid * _PER
        lane = lax.iota(jnp.int32, 16)
        thr = jnp.float32(_NMS_IOU_THRESHOLD)
        neg = jnp.float32(-1e9)
        inv = jnp.float32(1.0 / 512.0)
        big = jnp.int32(2 ** 30)
        sidv = jnp.broadcast_to(sid, (16,))
        negv = jnp.broadcast_to(neg, (16,))

        for src, dst in ((tyh, s0), (txh, s1), (thh, s2), (twh, s3),
                         (cbh, s4), (cfh, s5), (yah, s6), (xah, s7),
                         (hah, s8), (wah, s9)):
            pltpu.sync_copy(src.at[pl.ds(base, _PER)], dst)

        bv = jnp.full((16,), -jnp.inf, jnp.float32)
        bi = jnp.zeros((16,), jnp.int32)
        for j in range(_NJ):
            sl = pl.ds(j * 16, 16)
            ty = s0[sl] / 10.0
            tx = s1[sl] / 10.0
            th = s2[sl] / 5.0
            tw = s3[sl] / 5.0
            ya = s6[sl]
            xa = s7[sl]
            ha = s8[sl]
            wa = s9[sl]
            yc = ty * ha + ya
            xc = tx * wa + xa
            h = jnp.exp(th) * ha
            w = jnp.exp(tw) * wa
            ymin = jnp.clip(yc - h / 2.0, 0.0, 512.0)
            xmin = jnp.clip(xc - w / 2.0, 0.0, 512.0)
            ymax = jnp.clip(yc + h / 2.0, 0.0, 512.0)
            xmax = jnp.clip(xc + w / 2.0, 0.0, 512.0)
            cb = s4[sl]
            cf = s5[sl]
            mx = jnp.maximum(cb, cf)
            eb = jnp.exp(cb - mx)
            ef = jnp.exp(cf - mx)
            s = ef / (eb + ef)
            by_ref[sl] = ymin
            bx_ref[sl] = xmin
            ey_ref[sl] = ymax
            ex_ref[sl] = xmax
            ar_ref[sl] = (jnp.maximum(ymax - ymin, 0.0)
                          * jnp.maximum(xmax - xmin, 0.0))
            sc_ref[sl] = s
            linj = lane + (base + j * 16)
            better = s > bv
            bv = jnp.where(better, s, bv)
            bi = jnp.where(better, linj, bi)

        def step(t, carry):
            bv, bi = carry
            m_loc = jnp.max(bv)
            gl = jnp.min(jnp.where(bv == m_loc, bi, big))
            idxv = jnp.broadcast_to(gl - base, (16,))
            y0 = plsc.load_gather(by_ref, [idxv])
            x0 = plsc.load_gather(bx_ref, [idxv])
            y1 = plsc.load_gather(ey_ref, [idxv])
            x1 = plsc.load_gather(ex_ref, [idxv])
            pub = (jnp.where(lane == 0, m_loc, 0.0)
                   + jnp.where(lane == 1, y0, 0.0)
                   + jnp.where(lane == 2, x0, 0.0)
                   + jnp.where(lane == 3, y1, 0.0)
                   + jnp.where(lane == 4, x1, 0.0)
                   + jnp.where(lane == 5, gl.astype(jnp.float32), 0.0))
            pub_ref[...] = pub
            slot = lax.rem(t, 2)
            pltpu.sync_copy(pub_ref,
                            shared_ref.at[pl.ds(slot * 256 + sid * 16, 16)])
            plsc.subcore_barrier()
            pltpu.sync_copy(shared_ref.at[pl.ds(slot * 256, 256)], allc_ref)

            rows = lane * 16
            s16 = plsc.load_gather(allc_ref, [rows])
            y0a = plsc.load_gather(allc_ref, [rows + 1])
            x0a = plsc.load_gather(allc_ref, [rows + 2])
            y1a = plsc.load_gather(allc_ref, [rows + 3])
            x1a = plsc.load_gather(allc_ref, [rows + 4])
            i16 = plsc.load_gather(allc_ref, [rows + 5]).astype(jnp.int32)
            m = jnp.max(s16)
            g = jnp.min(jnp.where(s16 == m, i16, big))
            selg = i16 == g
            by0 = jnp.max(jnp.where(selg, y0a, -1.0))
            by1 = jnp.max(jnp.where(selg, x0a, -1.0))
            by2 = jnp.max(jnp.where(selg, y1a, -1.0))
            by3 = jnp.max(jnp.where(selg, x1a, -1.0))

            validv = jnp.broadcast_to(m, (16,)) > 0.0
            row = (jnp.where(lane == 0, by0, 0.0)
                   + jnp.where(lane == 1, by1, 0.0)
                   + jnp.where(lane == 2, by2, 0.0)
                   + jnp.where(lane == 3, by3, 0.0))
            row = jnp.where(validv, row, 0.0) * inv
            outv_ref[pl.ds(t * 16, 16)] = row

            gv = jnp.broadcast_to(g, (16,))
            killmask = (gv >= jnp.broadcast_to(base, (16,))) & \
                (gv < jnp.broadcast_to(base + _PER, (16,))) & (lane == 0)
            kidx = jnp.clip(gv - jnp.broadcast_to(base, (16,)), 0, _PER - 1)
            plsc.store_scatter(sc_ref, [kidx], negv, mask=killmask)

            area_a = (jnp.maximum(by2 - by0, 0.0)
                      * jnp.maximum(by3 - by1, 0.0))
            nbv = jnp.full((16,), -jnp.inf, jnp.float32)
            nbj = jnp.zeros((16,), jnp.int32)
            for j in range(_NJ):
                sl = pl.ds(j * 16, 16)
                ymin = by_ref[sl]
                xmin = bx_ref[sl]
                ymax = ey_ref[sl]
                xmax = ex_ref[sl]
                iy1 = jnp.maximum(by0, ymin)
                ix1 = jnp.maximum(by1, xmin)
                iy2 = jnp.minimum(by2, ymax)
                ix2 = jnp.minimum(by3, xmax)
                inter = (jnp.maximum(iy2 - iy1, 0.0)
                         * jnp.maximum(ix2 - ix1, 0.0))
                union = area_a + ar_ref[sl] - inter
                iou = inter / jnp.maximum(union, 1e-8)
                ns = jnp.where(iou > thr, negv, sc_ref[sl])
                sc_ref[sl] = ns
                better = ns > nbv
                nbv = jnp.where(better, ns, nbv)
                nbj = jnp.where(better, jnp.broadcast_to(j, (16,)), nbj)
            return (nbv, lane + (base + nbj * 16))

        lax.fori_loop(0, _MAX_PROPOSALS, step, (bv, bi))

        @pl.when(sid == 0)
        def _write_out():
            pltpu.sync_copy(outv_ref, outh)


def kernel(preprocessed_inputs, box_encodings, class_predictions_with_background,
           rpn_box_predictor_features, rpn_features_to_crop):
    del preprocessed_inputs, rpn_box_predictor_features, rpn_features_to_crop
    shape = (_ROWS, _COLS)
    enc = box_encodings[0]
    tyc = enc[:, 0].reshape(shape)
    txc = enc[:, 1].reshape(shape)
    thc = enc[:, 2].reshape(shape)
    twc = enc[:, 3].reshape(shape)
    cls = class_predictions_with_background[0]
    cb = cls[:, 0].reshape(shape)
    cf = cls[:, 1].reshape(shape)
    ya, xa, ha, wa = _anchor_planes(32, 32)
    f32 = jnp.float32

    mesh = plsc.VectorSubcoreMesh(core_axis_name="c", subcore_axis_name="s")
    run = functools.partial(
        pl.kernel,
        mesh=mesh,
        compiler_params=pltpu.CompilerParams(needs_layout_passes=False),
        out_type=jax.ShapeDtypeStruct((_MAX_PROPOSALS * 16,), f32),
        scratch_types=(
            [pltpu.VMEM((_PER,), f32) for _ in range(10)]
            + [pltpu.VMEM((_PER,), f32) for _ in range(6)]
            + [pltpu.VMEM((16,), f32),
               pltpu.VMEM((256,), f32),
               pltpu.VMEM((_MAX_PROPOSALS * 16,), f32),
               pltpu.VMEM_SHARED((512,), f32)]),
    )
    out = run(_sc_nms_body)(
        tyc.reshape(_N), txc.reshape(_N), thc.reshape(_N), twc.reshape(_N),
        cb.reshape(_N), cf.reshape(_N), ya.reshape(_N), xa.reshape(_N),
        ha.reshape(_N), wa.reshape(_N))
    return out.reshape(_MAX_PROPOSALS, 16)[:, :4][None]


# speedup vs baseline: 1.2670x; 1.0511x over previous
import functools
import numpy as np
import jax
import jax.numpy as jnp
from jax import lax
from jax.experimental import pallas as pl
from jax.experimental.pallas import tpu as pltpu
from jax.experimental.pallas import tpu_sc as plsc

_SCALES = (0.25, 0.5, 1.0, 2.0)
_ASPECT_RATIOS = (0.5, 1.0, 2.0)
_ANCHOR_STRIDE = (16, 16)
_MAX_PROPOSALS = 100
_NMS_IOU_THRESHOLD = 0.699999988079
_BASE_ANCHOR_SIZE = 256.0

_N = 12288
_ROWS, _COLS = 96, 128
_NSUB = 16
_PER = _N // _NSUB
_NJ = _PER // 16


def _anchor_planes(Hf, Wf):
    ys = (np.arange(Hf, dtype=np.float32) + 0.5) * _ANCHOR_STRIDE[0]
    xs = (np.arange(Wf, dtype=np.float32) + 0.5) * _ANCHOR_STRIDE[1]
    sc, ar = np.meshgrid(np.array(_SCALES, np.float32),
                         np.array(_ASPECT_RATIOS, np.float32), indexing='ij')
    sc = sc.reshape(-1)
    ar = ar.reshape(-1)
    ha = sc * _BASE_ANCHOR_SIZE / np.sqrt(ar)
    wa = sc * _BASE_ANCHOR_SIZE * np.sqrt(ar)
    A = ha.shape[0]
    yy, xx = np.meshgrid(ys, xs, indexing='ij')
    ycent = np.repeat(yy.reshape(-1), A)
    xcent = np.repeat(xx.reshape(-1), A)
    hh = np.tile(ha, Hf * Wf)
    ww = np.tile(wa, Hf * Wf)
    shape = (_ROWS, _COLS)
    return (jnp.asarray(ycent.reshape(shape)), jnp.asarray(xcent.reshape(shape)),
            jnp.asarray(hh.reshape(shape)), jnp.asarray(ww.reshape(shape)))


def _decode_body(tyr, txr, thr_, twr, cbr, cfr, yar, xar, har, war,
                 ymin_o, xmin_o, ymax_o, xmax_o, area_o, sc_o):
    ya = yar[:]
    xa = xar[:]
    ha = har[:]
    wa = war[:]
    ty = tyr[:] / 10.0
    tx = txr[:] / 10.0
    th = thr_[:] / 5.0
    tw = twr[:] / 5.0
    ycenter = ty * ha + ya
    xcenter = tx * wa + xa
    h = jnp.exp(th) * ha
    w = jnp.exp(tw) * wa
    ymin = jnp.clip(ycenter - h / 2.0, 0.0, 512.0)
    xmin = jnp.clip(xcenter - w / 2.0, 0.0, 512.0)
    ymax = jnp.clip(ycenter + h / 2.0, 0.0, 512.0)
    xmax = jnp.clip(xcenter + w / 2.0, 0.0, 512.0)
    cb = cbr[:]
    cf = cfr[:]
    mx = jnp.maximum(cb, cf)
    eb = jnp.exp(cb - mx)
    ef = jnp.exp(cf - mx)
    ymin_o[...] = ymin
    xmin_o[...] = xmin
    ymax_o[...] = ymax
    xmax_o[...] = xmax
    area_o[...] = (jnp.maximum(ymax - ymin, 0.0)
                   * jnp.maximum(xmax - xmin, 0.0))
    sc_o[...] = ef / (eb + ef)


def _sc_nms_body(byh, bxh, eyh, exh, arh, sch, outh,
                 by_ref, bx_ref, ey_ref, ex_ref, ar_ref, sc_ref,
                 pub_ref, allc_ref, outv_ref, shared_ref):
    cid = lax.axis_index("c")
    sid = lax.axis_index("s")

    @pl.when(cid == 0)
    def _core0():
        base = sid * _PER
        lane = lax.iota(jnp.int32, 16)
        thr = jnp.float32(_NMS_IOU_THRESHOLD)
        neg = jnp.float32(-1e9)
        inv = jnp.float32(1.0 / 512.0)
        big = jnp.int32(2 ** 30)
        negv = jnp.broadcast_to(neg, (16,))

        for src, dst in ((byh, by_ref), (bxh, bx_ref), (eyh, ey_ref),
                         (exh, ex_ref), (arh, ar_ref), (sch, sc_ref)):
            pltpu.sync_copy(src.at[pl.ds(base, _PER)], dst)

        bv = jnp.full((16,), -jnp.inf, jnp.float32)
        bi = jnp.zeros((16,), jnp.int32)
        for j in range(_NJ):
            s = sc_ref[pl.ds(j * 16, 16)]
            linj = lane + (base + j * 16)
            better = s > bv
            bv = jnp.where(better, s, bv)
            bi = jnp.where(better, linj, bi)

        def step(t, carry):
            bv, bi = carry
            m_loc = jnp.max(bv)
            gl = jnp.min(jnp.where(bv == m_loc, bi, big))
            idxv = jnp.broadcast_to(gl - base, (16,))
            y0 = plsc.load_gather(by_ref, [idxv])
            x0 = plsc.load_gather(bx_ref, [idxv])
            y1 = plsc.load_gather(ey_ref, [idxv])
            x1 = plsc.load_gather(ex_ref, [idxv])
            pub = (jnp.where(lane == 0, m_loc, 0.0)
                   + jnp.where(lane == 1, y0, 0.0)
                   + jnp.where(lane == 2, x0, 0.0)
                   + jnp.where(lane == 3, y1, 0.0)
                   + jnp.where(lane == 4, x1, 0.0)
                   + jnp.where(lane == 5, gl.astype(jnp.float32), 0.0))
            pub_ref[...] = pub
            slot = lax.rem(t, 2)
            pltpu.sync_copy(pub_ref,
                            shared_ref.at[pl.ds(slot * 256 + sid * 16, 16)])
            plsc.subcore_barrier()
            pltpu.sync_copy(shared_ref.at[pl.ds(slot * 256, 256)], allc_ref)

            rows = lane * 16
            s16 = plsc.load_gather(allc_ref, [rows])
            y0a = plsc.load_gather(allc_ref, [rows + 1])
            x0a = plsc.load_gather(allc_ref, [rows + 2])
            y1a = plsc.load_gather(allc_ref, [rows + 3])
            x1a = plsc.load_gather(allc_ref, [rows + 4])
            i16 = plsc.load_gather(allc_ref, [rows + 5]).astype(jnp.int32)
            m = jnp.max(s16)
            g = jnp.min(jnp.where(s16 == m, i16, big))
            selg = i16 == g
            by0 = jnp.max(jnp.where(selg, y0a, -1.0))
            by1 = jnp.max(jnp.where(selg, x0a, -1.0))
            by2 = jnp.max(jnp.where(selg, y1a, -1.0))
            by3 = jnp.max(jnp.where(selg, x1a, -1.0))

            validv = jnp.broadcast_to(m, (16,)) > 0.0
            row = (jnp.where(lane == 0, by0, 0.0)
                   + jnp.where(lane == 1, by1, 0.0)
                   + jnp.where(lane == 2, by2, 0.0)
                   + jnp.where(lane == 3, by3, 0.0))
            row = jnp.where(validv, row, 0.0) * inv
            outv_ref[pl.ds(t * 16, 16)] = row

            gv = jnp.broadcast_to(g, (16,))
            killmask = (gv >= jnp.broadcast_to(base, (16,))) & \
                (gv < jnp.broadcast_to(base + _PER, (16,))) & (lane == 0)
            kidx = jnp.clip(gv - jnp.broadcast_to(base, (16,)), 0, _PER - 1)
            plsc.store_scatter(sc_ref, [kidx], negv, mask=killmask)

            area_a = (jnp.maximum(by2 - by0, 0.0)
                      * jnp.maximum(by3 - by1, 0.0))
            nbv = jnp.full((16,), -jnp.inf, jnp.float32)
            nbj = jnp.zeros((16,), jnp.int32)
            for j in range(_NJ):
                sl = pl.ds(j * 16, 16)
                ymin = by_ref[sl]
                xmin = bx_ref[sl]
                ymax = ey_ref[sl]
                xmax = ex_ref[sl]
                iy1 = jnp.maximum(by0, ymin)
                ix1 = jnp.maximum(by1, xmin)
                iy2 = jnp.minimum(by2, ymax)
                ix2 = jnp.minimum(by3, xmax)
                inter = (jnp.maximum(iy2 - iy1, 0.0)
                         * jnp.maximum(ix2 - ix1, 0.0))
                union = area_a + ar_ref[sl] - inter
                iou = inter / jnp.maximum(union, 1e-8)
                ns = jnp.where(iou > thr, negv, sc_ref[sl])
                sc_ref[sl] = ns
                better = ns > nbv
                nbv = jnp.where(better, ns, nbv)
                nbj = jnp.where(better, jnp.broadcast_to(j, (16,)), nbj)
            return (nbv, lane + (base + nbj * 16))

        lax.fori_loop(0, _MAX_PROPOSALS, step, (bv, bi))

        @pl.when(sid == 0)
        def _write_out():
            pltpu.sync_copy(outv_ref, outh)


def kernel(preprocessed_inputs, box_encodings, class_predictions_with_background,
           rpn_box_predictor_features, rpn_features_to_crop):
    del preprocessed_inputs, rpn_box_predictor_features, rpn_features_to_crop
    shape = (_ROWS, _COLS)
    enc = box_encodings[0]
    tyc = enc[:, 0].reshape(shape)
    txc = enc[:, 1].reshape(shape)
    thc = enc[:, 2].reshape(shape)
    twc = enc[:, 3].reshape(shape)
    cls = class_predictions_with_background[0]
    cb = cls[:, 0].reshape(shape)
    cf = cls[:, 1].reshape(shape)
    ya, xa, ha, wa = _anchor_planes(32, 32)
    f32 = jnp.float32
    plane = jax.ShapeDtypeStruct(shape, f32)
    ymin, xmin, ymax, xmax, area, scores = pl.pallas_call(
        _decode_body,
        out_shape=[plane] * 6,
    )(tyc, txc, thc, twc, cb, cf, ya, xa, ha, wa)

    mesh = plsc.VectorSubcoreMesh(core_axis_name="c", subcore_axis_name="s")
    run = functools.partial(
        pl.kernel,
        mesh=mesh,
        compiler_params=pltpu.CompilerParams(needs_layout_passes=False),
        out_type=jax.ShapeDtypeStruct((_MAX_PROPOSALS * 16,), f32),
        scratch_types=(
            [pltpu.VMEM((_PER,), f32) for _ in range(6)]
            + [pltpu.VMEM((16,), f32),
               pltpu.VMEM((256,), f32),
               pltpu.VMEM((_MAX_PROPOSALS * 16,), f32),
               pltpu.VMEM_SHARED((512,), f32)]),
    )
    out = run(_sc_nms_body)(
        ymin.reshape(_N), xmin.reshape(_N), ymax.reshape(_N),
        xmax.reshape(_N), area.reshape(_N), scores.reshape(_N))
    return out.reshape(_MAX_PROPOSALS, 16)[:, :4][None]


# speedup vs baseline: 1.3054x; 1.0303x over previous
import functools
import numpy as np
import jax
import jax.numpy as jnp
from jax import lax
from jax.experimental import pallas as pl
from jax.experimental.pallas import tpu as pltpu
from jax.experimental.pallas import tpu_sc as plsc

_SCALES = (0.25, 0.5, 1.0, 2.0)
_ASPECT_RATIOS = (0.5, 1.0, 2.0)
_ANCHOR_STRIDE = (16, 16)
_MAX_PROPOSALS = 100
_NMS_IOU_THRESHOLD = 0.699999988079
_BASE_ANCHOR_SIZE = 256.0

_N = 12288
_ROWS, _COLS = 96, 128
_NSUB = 16
_PER = _N // _NSUB
_NJ = _PER // 16


def _anchor_planes(Hf, Wf):
    ys = (np.arange(Hf, dtype=np.float32) + 0.5) * _ANCHOR_STRIDE[0]
    xs = (np.arange(Wf, dtype=np.float32) + 0.5) * _ANCHOR_STRIDE[1]
    sc, ar = np.meshgrid(np.array(_SCALES, np.float32),
                         np.array(_ASPECT_RATIOS, np.float32), indexing='ij')
    sc = sc.reshape(-1)
    ar = ar.reshape(-1)
    ha = sc * _BASE_ANCHOR_SIZE / np.sqrt(ar)
    wa = sc * _BASE_ANCHOR_SIZE * np.sqrt(ar)
    A = ha.shape[0]
    yy, xx = np.meshgrid(ys, xs, indexing='ij')
    ycent = np.repeat(yy.reshape(-1), A)
    xcent = np.repeat(xx.reshape(-1), A)
    hh = np.tile(ha, Hf * Wf)
    ww = np.tile(wa, Hf * Wf)
    shape = (_ROWS, _COLS)
    return (jnp.asarray(ycent.reshape(shape)), jnp.asarray(xcent.reshape(shape)),
            jnp.asarray(hh.reshape(shape)), jnp.asarray(ww.reshape(shape)))


def _decode_body(tyr, txr, thr_, twr, cbr, cfr, yar, xar, har, war,
                 ymin_o, xmin_o, ymax_o, xmax_o, area_o, sc_o):
    ya = yar[:]
    xa = xar[:]
    ha = har[:]
    wa = war[:]
    ty = tyr[:] / 10.0
    tx = txr[:] / 10.0
    th = thr_[:] / 5.0
    tw = twr[:] / 5.0
    ycenter = ty * ha + ya
    xcenter = tx * wa + xa
    h = jnp.exp(th) * ha
    w = jnp.exp(tw) * wa
    ymin = jnp.clip(ycenter - h / 2.0, 0.0, 512.0)
    xmin = jnp.clip(xcenter - w / 2.0, 0.0, 512.0)
    ymax = jnp.clip(ycenter + h / 2.0, 0.0, 512.0)
    xmax = jnp.clip(xcenter + w / 2.0, 0.0, 512.0)
    cb = cbr[:]
    cf = cfr[:]
    mx = jnp.maximum(cb, cf)
    eb = jnp.exp(cb - mx)
    ef = jnp.exp(cf - mx)
    ymin_o[...] = ymin
    xmin_o[...] = xmin
    ymax_o[...] = ymax
    xmax_o[...] = xmax
    area_o[...] = (jnp.maximum(ymax - ymin, 0.0)
                   * jnp.maximum(xmax - xmin, 0.0))
    sc_o[...] = ef / (eb + ef)


def _sc_nms_body(byh, bxh, eyh, exh, arh, sch, outh,
                 by_ref, bx_ref, ey_ref, ex_ref, ar_ref, sc_ref,
                 pub_ref, allc_ref, outv_ref, shared_ref):
    cid = lax.axis_index("c")
    sid = lax.axis_index("s")

    @pl.when(cid == 0)
    def _core0():
        base = sid * _PER
        lane = lax.iota(jnp.int32, 16)
        thr = jnp.float32(_NMS_IOU_THRESHOLD)
        neg = jnp.float32(-1e9)
        inv = jnp.float32(1.0 / 512.0)
        big = jnp.int32(2 ** 30)
        negv = jnp.broadcast_to(neg, (16,))

        for src, dst in ((byh, by_ref), (bxh, bx_ref), (eyh, ey_ref),
                         (exh, ex_ref), (arh, ar_ref), (sch, sc_ref)):
            pltpu.sync_copy(src.at[pl.ds(base, _PER)], dst)

        bv = jnp.full((16,), -jnp.inf, jnp.float32)
        bi = jnp.zeros((16,), jnp.int32)
        for j in range(_NJ):
            s = sc_ref[pl.ds(j * 16, 16)]
            linj = lane + (base + j * 16)
            better = s > bv
            bv = jnp.where(better, s, bv)
            bi = jnp.where(better, linj, bi)

        def step(t, carry):
            bv, bi = carry
            m_loc = jnp.max(bv)
            gl = jnp.min(jnp.where(bv == m_loc, bi, big))
            idxv = jnp.broadcast_to(gl - base, (16,))
            y0 = plsc.load_gather(by_ref, [idxv])
            x0 = plsc.load_gather(bx_ref, [idxv])
            y1 = plsc.load_gather(ey_ref, [idxv])
            x1 = plsc.load_gather(ex_ref, [idxv])
            pub = (jnp.where(lane == 0, m_loc, 0.0)
                   + jnp.where(lane == 1, y0, 0.0)
                   + jnp.where(lane == 2, x0, 0.0)
                   + jnp.where(lane == 3, y1, 0.0)
                   + jnp.where(lane == 4, x1, 0.0)
                   + jnp.where(lane == 5, gl.astype(jnp.float32), 0.0))
            pub_ref[...] = pub
            slot = lax.rem(t, 2)
            pltpu.sync_copy(pub_ref,
                            shared_ref.at[pl.ds(slot * 256 + sid * 16, 16)])
            plsc.subcore_barrier()
            pltpu.sync_copy(shared_ref.at[pl.ds(slot * 256, 256)], allc_ref)

            s16 = plsc.load_gather(allc_ref, [lane * 16])
            m = jnp.max(s16)
            fi = jnp.broadcast_to(plsc.all_reduce_ffs(s16 == m), (16,)) * 16
            by0 = plsc.load_gather(allc_ref, [fi + 1])
            by1 = plsc.load_gather(allc_ref, [fi + 2])
            by2 = plsc.load_gather(allc_ref, [fi + 3])
            by3 = plsc.load_gather(allc_ref, [fi + 4])
            g = plsc.load_gather(allc_ref, [fi + 5]).astype(jnp.int32)

            validv = jnp.broadcast_to(m, (16,)) > 0.0
            row = (jnp.where(lane == 0, by0, 0.0)
                   + jnp.where(lane == 1, by1, 0.0)
                   + jnp.where(lane == 2, by2, 0.0)
                   + jnp.where(lane == 3, by3, 0.0))
            row = jnp.where(validv, row, 0.0) * inv
            outv_ref[pl.ds(t * 16, 16)] = row

            gv = jnp.broadcast_to(g, (16,))
            killmask = (gv >= jnp.broadcast_to(base, (16,))) & \
                (gv < jnp.broadcast_to(base + _PER, (16,))) & (lane == 0)
            kidx = jnp.clip(gv - jnp.broadcast_to(base, (16,)), 0, _PER - 1)
            plsc.store_scatter(sc_ref, [kidx], negv, mask=killmask)

            area_a = (jnp.maximum(by2 - by0, 0.0)
                      * jnp.maximum(by3 - by1, 0.0))
            nbv = jnp.full((16,), -jnp.inf, jnp.float32)
            nbj = jnp.zeros((16,), jnp.int32)
            for j in range(_NJ):
                sl = pl.ds(j * 16, 16)
                ymin = by_ref[sl]
                xmin = bx_ref[sl]
                ymax = ey_ref[sl]
                xmax = ex_ref[sl]
                iy1 = jnp.maximum(by0, ymin)
                ix1 = jnp.maximum(by1, xmin)
                iy2 = jnp.minimum(by2, ymax)
                ix2 = jnp.minimum(by3, xmax)
                inter = (jnp.maximum(iy2 - iy1, 0.0)
                         * jnp.maximum(ix2 - ix1, 0.0))
                union = area_a + ar_ref[sl] - inter
                iou = inter / jnp.maximum(union, 1e-8)
                ns = jnp.where(iou > thr, negv, sc_ref[sl])
                sc_ref[sl] = ns
                better = ns > nbv
                nbv = jnp.where(better, ns, nbv)
                nbj = jnp.where(better, jnp.broadcast_to(j, (16,)), nbj)
            return (nbv, lane + (base + nbj * 16))

        lax.fori_loop(0, _MAX_PROPOSALS, step, (bv, bi))

        @pl.when(sid == 0)
        def _write_out():
            pltpu.sync_copy(outv_ref, outh)


def kernel(preprocessed_inputs, box_encodings, class_predictions_with_background,
           rpn_box_predictor_features, rpn_features_to_crop):
    del preprocessed_inputs, rpn_box_predictor_features, rpn_features_to_crop
    shape = (_ROWS, _COLS)
    enc = box_encodings[0]
    tyc = enc[:, 0].reshape(shape)
    txc = enc[:, 1].reshape(shape)
    thc = enc[:, 2].reshape(shape)
    twc = enc[:, 3].reshape(shape)
    cls = class_predictions_with_background[0]
    cb = cls[:, 0].reshape(shape)
    cf = cls[:, 1].reshape(shape)
    ya, xa, ha, wa = _anchor_planes(32, 32)
    f32 = jnp.float32
    plane = jax.ShapeDtypeStruct(shape, f32)
    ymin, xmin, ymax, xmax, area, scores = pl.pallas_call(
        _decode_body,
        out_shape=[plane] * 6,
    )(tyc, txc, thc, twc, cb, cf, ya, xa, ha, wa)

    mesh = plsc.VectorSubcoreMesh(core_axis_name="c", subcore_axis_name="s")
    run = functools.partial(
        pl.kernel,
        mesh=mesh,
        compiler_params=pltpu.CompilerParams(needs_layout_passes=False),
        out_type=jax.ShapeDtypeStruct((_MAX_PROPOSALS * 16,), f32),
        scratch_types=(
            [pltpu.VMEM((_PER,), f32) for _ in range(6)]
            + [pltpu.VMEM((16,), f32),
               pltpu.VMEM((256,), f32),
               pltpu.VMEM((_MAX_PROPOSALS * 16,), f32),
               pltpu.VMEM_SHARED((512,), f32)]),
    )
    out = run(_sc_nms_body)(
        ymin.reshape(_N), xmin.reshape(_N), ymax.reshape(_N),
        xmax.reshape(_N), area.reshape(_N), scores.reshape(_N))
    return out.reshape(_MAX_PROPOSALS, 16)[:, :4][None]


# speedup vs baseline: 1.3284x; 1.0176x over previous
import functools
import numpy as np
import jax
import jax.numpy as jnp
from jax import lax
from jax.experimental import pallas as pl
from jax.experimental.pallas import tpu as pltpu
from jax.experimental.pallas import tpu_sc as plsc

_SCALES = (0.25, 0.5, 1.0, 2.0)
_ASPECT_RATIOS = (0.5, 1.0, 2.0)
_ANCHOR_STRIDE = (16, 16)
_MAX_PROPOSALS = 100
_NMS_IOU_THRESHOLD = 0.699999988079
_BASE_ANCHOR_SIZE = 256.0

_N = 12288
_ROWS, _COLS = 96, 128
_NSUB = 16
_PER = _N // _NSUB
_NJ = _PER // 16


def _anchor_planes(Hf, Wf):
    ys = (np.arange(Hf, dtype=np.float32) + 0.5) * _ANCHOR_STRIDE[0]
    xs = (np.arange(Wf, dtype=np.float32) + 0.5) * _ANCHOR_STRIDE[1]
    sc, ar = np.meshgrid(np.array(_SCALES, np.float32),
                         np.array(_ASPECT_RATIOS, np.float32), indexing='ij')
    sc = sc.reshape(-1)
    ar = ar.reshape(-1)
    ha = sc * _BASE_ANCHOR_SIZE / np.sqrt(ar)
    wa = sc * _BASE_ANCHOR_SIZE * np.sqrt(ar)
    A = ha.shape[0]
    yy, xx = np.meshgrid(ys, xs, indexing='ij')
    ycent = np.repeat(yy.reshape(-1), A)
    xcent = np.repeat(xx.reshape(-1), A)
    hh = np.tile(ha, Hf * Wf)
    ww = np.tile(wa, Hf * Wf)
    shape = (_ROWS, _COLS)
    return (jnp.asarray(ycent.reshape(shape)), jnp.asarray(xcent.reshape(shape)),
            jnp.asarray(hh.reshape(shape)), jnp.asarray(ww.reshape(shape)))


def _decode_body(tyr, txr, thr_, twr, cbr, cfr, yar, xar, har, war, dec_o):
    ya = yar[:]
    xa = xar[:]
    ha = har[:]
    wa = war[:]
    ty = tyr[:] / 10.0
    tx = txr[:] / 10.0
    th = thr_[:] / 5.0
    tw = twr[:] / 5.0
    ycenter = ty * ha + ya
    xcenter = tx * wa + xa
    h = jnp.exp(th) * ha
    w = jnp.exp(tw) * wa
    ymin = jnp.clip(ycenter - h / 2.0, 0.0, 512.0)
    xmin = jnp.clip(xcenter - w / 2.0, 0.0, 512.0)
    ymax = jnp.clip(ycenter + h / 2.0, 0.0, 512.0)
    xmax = jnp.clip(xcenter + w / 2.0, 0.0, 512.0)
    cb = cbr[:]
    cf = cfr[:]
    mx = jnp.maximum(cb, cf)
    eb = jnp.exp(cb - mx)
    ef = jnp.exp(cf - mx)
    dec_o[0] = ymin
    dec_o[1] = xmin
    dec_o[2] = ymax
    dec_o[3] = xmax
    dec_o[4] = (jnp.maximum(ymax - ymin, 0.0)
                * jnp.maximum(xmax - xmin, 0.0))
    dec_o[5] = ef / (eb + ef)


def _sc_nms_body(dech, outh,
                 by_ref, bx_ref, ey_ref, ex_ref, ar_ref, sc_ref,
                 pub_ref, allc_ref, outv_ref, shared_ref):
    cid = lax.axis_index("c")
    sid = lax.axis_index("s")

    @pl.when(cid == 0)
    def _core0():
        base = sid * _PER
        lane = lax.iota(jnp.int32, 16)
        thr = jnp.float32(_NMS_IOU_THRESHOLD)
        neg = jnp.float32(-1e9)
        inv = jnp.float32(1.0 / 512.0)
        big = jnp.int32(2 ** 30)
        negv = jnp.broadcast_to(neg, (16,))

        for f, dst in enumerate((by_ref, bx_ref, ey_ref, ex_ref,
                                 ar_ref, sc_ref)):
            pltpu.sync_copy(dech.at[pl.ds(f * _N + base, _PER)], dst)

        bv = jnp.full((16,), -jnp.inf, jnp.float32)
        bi = jnp.zeros((16,), jnp.int32)
        for j in range(_NJ):
            s = sc_ref[pl.ds(j * 16, 16)]
            linj = lane + (base + j * 16)
            better = s > bv
            bv = jnp.where(better, s, bv)
            bi = jnp.where(better, linj, bi)

        def step(t, carry):
            bv, bi = carry
            m_loc = jnp.max(bv)
            gl = jnp.min(jnp.where(bv == m_loc, bi, big))
            idxv = jnp.broadcast_to(gl - base, (16,))
            y0 = plsc.load_gather(by_ref, [idxv])
            x0 = plsc.load_gather(bx_ref, [idxv])
            y1 = plsc.load_gather(ey_ref, [idxv])
            x1 = plsc.load_gather(ex_ref, [idxv])
            pub = (jnp.where(lane == 0, m_loc, 0.0)
                   + jnp.where(lane == 1, y0, 0.0)
                   + jnp.where(lane == 2, x0, 0.0)
                   + jnp.where(lane == 3, y1, 0.0)
                   + jnp.where(lane == 4, x1, 0.0)
                   + jnp.where(lane == 5, gl.astype(jnp.float32), 0.0))
            pub_ref[...] = pub
            slot = lax.rem(t, 2)
            pltpu.sync_copy(pub_ref,
                            shared_ref.at[pl.ds(slot * 256 + sid * 16, 16)])
            plsc.subcore_barrier()
            pltpu.sync_copy(shared_ref.at[pl.ds(slot * 256, 256)], allc_ref)

            s16 = plsc.load_gather(allc_ref, [lane * 16])
            m = jnp.max(s16)
            fi = jnp.broadcast_to(plsc.all_reduce_ffs(s16 == m), (16,)) * 16
            by0 = plsc.load_gather(allc_ref, [fi + 1])
            by1 = plsc.load_gather(allc_ref, [fi + 2])
            by2 = plsc.load_gather(allc_ref, [fi + 3])
            by3 = plsc.load_gather(allc_ref, [fi + 4])
            g = plsc.load_gather(allc_ref, [fi + 5]).astype(jnp.int32)

            validv = jnp.broadcast_to(m, (16,)) > 0.0
            row = (jnp.where(lane == 0, by0, 0.0)
                   + jnp.where(lane == 1, by1, 0.0)
                   + jnp.where(lane == 2, by2, 0.0)
                   + jnp.where(lane == 3, by3, 0.0))
            row = jnp.where(validv, row, 0.0) * inv
            outv_ref[pl.ds(t * 16, 16)] = row

            gv = jnp.broadcast_to(g, (16,))
            killmask = (gv >= jnp.broadcast_to(base, (16,))) & \
                (gv < jnp.broadcast_to(base + _PER, (16,))) & (lane == 0)
            kidx = jnp.clip(gv - jnp.broadcast_to(base, (16,)), 0, _PER - 1)
            plsc.store_scatter(sc_ref, [kidx], negv, mask=killmask)

            area_a = (jnp.maximum(by2 - by0, 0.0)
                      * jnp.maximum(by3 - by1, 0.0))
            nbv = jnp.full((16,), -jnp.inf, jnp.float32)
            nbj = jnp.zeros((16,), jnp.int32)
            for j in range(_NJ):
                sl = pl.ds(j * 16, 16)
                ymin = by_ref[sl]
                xmin = bx_ref[sl]
                ymax = ey_ref[sl]
                xmax = ex_ref[sl]
                iy1 = jnp.maximum(by0, ymin)
                ix1 = jnp.maximum(by1, xmin)
                iy2 = jnp.minimum(by2, ymax)
                ix2 = jnp.minimum(by3, xmax)
                inter = (jnp.maximum(iy2 - iy1, 0.0)
                         * jnp.maximum(ix2 - ix1, 0.0))
                union = area_a + ar_ref[sl] - inter
                iou = inter / jnp.maximum(union, 1e-8)
                ns = jnp.where(iou > thr, negv, sc_ref[sl])
                sc_ref[sl] = ns
                better = ns > nbv
                nbv = jnp.where(better, ns, nbv)
                nbj = jnp.where(better, jnp.broadcast_to(j, (16,)), nbj)
            return (nbv, lane + (base + nbj * 16))

        lax.fori_loop(0, _MAX_PROPOSALS, step, (bv, bi))

        @pl.when(sid == 0)
        def _write_out():
            pltpu.sync_copy(outv_ref, outh)


def kernel(preprocessed_inputs, box_encodings, class_predictions_with_background,
           rpn_box_predictor_features, rpn_features_to_crop):
    del preprocessed_inputs, rpn_box_predictor_features, rpn_features_to_crop
    shape = (_ROWS, _COLS)
    enc = box_encodings[0]
    tyc = enc[:, 0].reshape(shape)
    txc = enc[:, 1].reshape(shape)
    thc = enc[:, 2].reshape(shape)
    twc = enc[:, 3].reshape(shape)
    cls = class_predictions_with_background[0]
    cb = cls[:, 0].reshape(shape)
    cf = cls[:, 1].reshape(shape)
    ya, xa, ha, wa = _anchor_planes(32, 32)
    f32 = jnp.float32
    dec = pl.pallas_call(
        _decode_body,
        out_shape=jax.ShapeDtypeStruct((6,) + shape, f32),
    )(tyc, txc, thc, twc, cb, cf, ya, xa, ha, wa)

    mesh = plsc.VectorSubcoreMesh(core_axis_name="c", subcore_axis_name="s")
    run = functools.partial(
        pl.kernel,
        mesh=mesh,
        compiler_params=pltpu.CompilerParams(needs_layout_passes=False),
        out_type=jax.ShapeDtypeStruct((_MAX_PROPOSALS * 16,), f32),
        scratch_types=(
            [pltpu.VMEM((_PER,), f32) for _ in range(6)]
            + [pltpu.VMEM((16,), f32),
               pltpu.VMEM((256,), f32),
               pltpu.VMEM((_MAX_PROPOSALS * 16,), f32),
               pltpu.VMEM_SHARED((512,), f32)]),
    )
    out = run(_sc_nms_body)(dec.reshape(6 * _N))
    return out.reshape(_MAX_PROPOSALS, 16)[:, :4][None]
